# baseline (speedup 1.0000x reference)
import functools

import jax
import jax.numpy as jnp
from jax import lax
from jax.experimental import pallas as pl
from jax.experimental.pallas import tpu as pltpu
from jax.experimental.pallas import tpu_sc as plsc

BATCH = 16384
DIM = 32
LANES = 16
NUM_CORES = 2
NUM_SUBCORES = 16
NUM_WORKERS = NUM_CORES * NUM_SUBCORES
B_PER_W = BATCH // NUM_WORKERS
GROUPS = B_PER_W // LANES

_MESH = plsc.VectorSubcoreMesh(core_axis_name="c", subcore_axis_name="s")

_CP = pltpu.CompilerParams(needs_layout_passes=False, use_tc_tiling_on_sc=False)


@functools.partial(
    pl.kernel,
    out_type=jax.ShapeDtypeStruct((BATCH,), jnp.float32),
    mesh=_MESH,
    compiler_params=_CP,
    scratch_types=[
        pltpu.VMEM((B_PER_W,), jnp.int32),
        pltpu.VMEM((B_PER_W,), jnp.int32),
        pltpu.VMEM((DIM, B_PER_W), jnp.float32),
        pltpu.VMEM((DIM, B_PER_W), jnp.float32),
        pltpu.VMEM((B_PER_W,), jnp.float32),
        pltpu.SemaphoreType.DMA,
        pltpu.SemaphoreType.DMA,
    ],
)
def _two_tower_sc(uids_hbm, iids_hbm, utabT_hbm, itabT_hbm, out_hbm,
                  uidx_v, iidx_v, ucols_v, icols_v, out_v, sem_u, sem_i):
    wid = lax.axis_index("s") * NUM_CORES + lax.axis_index("c")
    base = wid * B_PER_W

    pltpu.sync_copy(uids_hbm.at[pl.ds(base, B_PER_W)], uidx_v)
    pltpu.sync_copy(iids_hbm.at[pl.ds(base, B_PER_W)], iidx_v)

    copies = []
    for d in range(DIM):
        copies.append(
            pltpu.async_copy(utabT_hbm.at[d].at[uidx_v], ucols_v.at[d], sem_u))
        copies.append(
            pltpu.async_copy(itabT_hbm.at[d].at[iidx_v], icols_v.at[d], sem_i))
    for c in copies:
        c.wait()

    @pl.loop(0, GROUPS)
    def _(g):
        sl = pl.ds(g * LANES, LANES)
        acc = jnp.zeros((LANES,), jnp.float32)
        for d in range(DIM):
            acc = acc + ucols_v[d, sl] * icols_v[d, sl]
        out_v[sl] = acc

    pltpu.sync_copy(out_v, out_hbm.at[pl.ds(base, B_PER_W)])


def kernel(user_ids, item_ids, user_table, item_table):
    user_ids = user_ids.astype(jnp.int32)
    item_ids = item_ids.astype(jnp.int32)
    return _two_tower_sc(user_ids, item_ids, user_table.T, item_table.T)


# speedup vs baseline: 22.0915x; 22.0915x over previous
import functools

import jax
import jax.numpy as jnp
from jax import lax
from jax.experimental import pallas as pl
from jax.experimental.pallas import tpu as pltpu
from jax.experimental.pallas import tpu_sc as plsc

BATCH = 16384
DIM = 32
LANES = 16
NUM_CORES = 2
NUM_SUBCORES = 16
NUM_WORKERS = NUM_CORES * NUM_SUBCORES
B_PER_W = BATCH // NUM_WORKERS
DEPTH = 4

_MESH = plsc.VectorSubcoreMesh(core_axis_name="c", subcore_axis_name="s")
_CP = pltpu.CompilerParams(needs_layout_passes=False)


@functools.partial(
    pl.kernel,
    out_type=jax.ShapeDtypeStruct((BATCH,), jnp.float32),
    mesh=_MESH,
    compiler_params=_CP,
    scratch_types=[
        pltpu.VMEM((DEPTH, DIM, 128), jnp.float32),
        pltpu.VMEM((DEPTH, DIM, 128), jnp.float32),
        pltpu.VMEM((LANES, LANES), jnp.float32),
        pltpu.VMEM((B_PER_W,), jnp.float32),
        pltpu.VMEM((B_PER_W + LANES,), jnp.int32),
        pltpu.VMEM((B_PER_W + LANES,), jnp.int32),
        pltpu.SemaphoreType.DMA,
        pltpu.SemaphoreType.DMA,
        pltpu.SemaphoreType.DMA,
        pltpu.SemaphoreType.DMA,
        pltpu.SemaphoreType.DMA,
        pltpu.SemaphoreType.DMA,
        pltpu.SemaphoreType.DMA,
        pltpu.SemaphoreType.DMA,
    ],
)
def _two_tower_sc(uids_hbm, iids_hbm, utabT_hbm, itabT_hbm, out_hbm,
                  ublk_v, iblk_v, pbuf_v, out_v,
                  uids_v, iids_v, *sems):
    usem = sems[:DEPTH]
    isem = sems[DEPTH:]
    wid = lax.axis_index("s") * NUM_CORES + lax.axis_index("c")
    base = wid * B_PER_W

    pltpu.sync_copy(uids_hbm.at[pl.ds(base, B_PER_W)], uids_v.at[pl.ds(0, B_PER_W)])
    pltpu.sync_copy(iids_hbm.at[pl.ds(base, B_PER_W)], iids_v.at[pl.ds(0, B_PER_W)])

    def issue(j, r):
        uid = uids_v[pl.ds(j, LANES)][0]
        iid = iids_v[pl.ds(j, LANES)][0]
        ub = pl.multiple_of(uid & ~127, 128)
        ib = pl.multiple_of(iid & ~127, 128)
        pltpu.async_copy(utabT_hbm.at[:, pl.ds(ub, 128)], ublk_v.at[r], usem[r])
        pltpu.async_copy(itabT_hbm.at[:, pl.ds(ib, 128)], iblk_v.at[r], isem[r])

    for r in range(DEPTH):
        issue(r, r)

    iota = lax.iota(jnp.int32, LANES)
    niters = B_PER_W // DEPTH

    @pl.loop(0, niters)
    def _(jj):
        for r in range(DEPTH):
            j = jj * DEPTH + r
            pltpu.make_async_copy(
                utabT_hbm.at[:, pl.ds(0, 128)], ublk_v.at[r], usem[r]).wait()
            pltpu.make_async_copy(
                itabT_hbm.at[:, pl.ds(0, 128)], iblk_v.at[r], isem[r]).wait()

            ul = jnp.full((LANES,), uids_v[pl.ds(j, LANES)][0] & 127, jnp.int32)
            il = jnp.full((LANES,), iids_v[pl.ds(j, LANES)][0] & 127, jnp.int32)
            rr = jnp.full((LANES,), r, jnp.int32)
            u0 = plsc.load_gather(ublk_v, [rr, iota, ul])
            u1 = plsc.load_gather(ublk_v, [rr, iota + LANES, ul])
            i0 = plsc.load_gather(iblk_v, [rr, iota, il])
            i1 = plsc.load_gather(iblk_v, [rr, iota + LANES, il])
            p = u0 * i0 + u1 * i1

            row = (jj % 4) * DEPTH + r
            pbuf_v[row] = p

            @pl.when(jj < niters - 1)
            def _():
                issue(j + DEPTH, r)

            if r == DEPTH - 1:
                @pl.when(jj % 4 == 3)
                def _():
                    acc = jnp.zeros((LANES,), jnp.float32)
                    for d in range(LANES):
                        acc = acc + plsc.load_gather(
                            pbuf_v, [iota, jnp.full((LANES,), d, jnp.int32)])
                    g0 = (jj - 3) * DEPTH
                    out_v[pl.ds(g0, LANES)] = acc

    pltpu.sync_copy(out_v, out_hbm.at[pl.ds(base, B_PER_W)])


def kernel(user_ids, item_ids, user_table, item_table):
    user_ids = user_ids.astype(jnp.int32)
    item_ids = item_ids.astype(jnp.int32)
    return _two_tower_sc(user_ids, item_ids, user_table.T, item_table.T)


# speedup vs baseline: 22.6739x; 1.0264x over previous
import functools

import jax
import jax.numpy as jnp
from jax import lax
from jax.experimental import pallas as pl
from jax.experimental.pallas import tpu as pltpu
from jax.experimental.pallas import tpu_sc as plsc

BATCH = 16384
DIM = 32
LANES = 16
NUM_CORES = 2
NUM_SUBCORES = 16
NUM_WORKERS = NUM_CORES * NUM_SUBCORES
B_PER_W = BATCH // NUM_WORKERS
DEPTH = 8

GROUP_ITERS = LANES // DEPTH

_MESH = plsc.VectorSubcoreMesh(core_axis_name="c", subcore_axis_name="s")
_CP = pltpu.CompilerParams(needs_layout_passes=False)


@functools.partial(
    pl.kernel,
    out_type=jax.ShapeDtypeStruct((BATCH,), jnp.float32),
    mesh=_MESH,
    compiler_params=_CP,
    scratch_types=[
        pltpu.VMEM((DEPTH, DIM, 128), jnp.float32),
        pltpu.VMEM((DEPTH, DIM, 128), jnp.float32),
        pltpu.VMEM((LANES, LANES), jnp.float32),
        pltpu.VMEM((B_PER_W,), jnp.float32),
        pltpu.VMEM((B_PER_W + LANES,), jnp.int32),
        pltpu.VMEM((B_PER_W + LANES,), jnp.int32),
    ] + [pltpu.SemaphoreType.DMA] * (2 * DEPTH),
)
def _two_tower_sc(uids_hbm, iids_hbm, utabT_hbm, itabT_hbm, out_hbm,
                  ublk_v, iblk_v, pbuf_v, out_v,
                  uids_v, iids_v, *sems):
    usem = sems[:DEPTH]
    isem = sems[DEPTH:]
    wid = lax.axis_index("s") * NUM_CORES + lax.axis_index("c")
    base = wid * B_PER_W

    pltpu.sync_copy(uids_hbm.at[pl.ds(base, B_PER_W)], uids_v.at[pl.ds(0, B_PER_W)])
    pltpu.sync_copy(iids_hbm.at[pl.ds(base, B_PER_W)], iids_v.at[pl.ds(0, B_PER_W)])

    def issue(j, r):
        uid = uids_v[pl.ds(j, LANES)][0]
        iid = iids_v[pl.ds(j, LANES)][0]
        ub = pl.multiple_of(uid & ~127, 128)
        ib = pl.multiple_of(iid & ~127, 128)
        pltpu.async_copy(utabT_hbm.at[:, pl.ds(ub, 128)], ublk_v.at[r], usem[r])
        pltpu.async_copy(itabT_hbm.at[:, pl.ds(ib, 128)], iblk_v.at[r], isem[r])

    for r in range(DEPTH):
        issue(r, r)

    iota = lax.iota(jnp.int32, LANES)
    niters = B_PER_W // DEPTH

    @pl.loop(0, niters)
    def _(jj):
        for r in range(DEPTH):
            j = jj * DEPTH + r
            pltpu.make_async_copy(
                utabT_hbm.at[:, pl.ds(0, 128)], ublk_v.at[r], usem[r]).wait()
            pltpu.make_async_copy(
                itabT_hbm.at[:, pl.ds(0, 128)], iblk_v.at[r], isem[r]).wait()

            ul = jnp.full((LANES,), uids_v[pl.ds(j, LANES)][0] & 127, jnp.int32)
            il = jnp.full((LANES,), iids_v[pl.ds(j, LANES)][0] & 127, jnp.int32)
            rr = jnp.full((LANES,), r, jnp.int32)
            u0 = plsc.load_gather(ublk_v, [rr, iota, ul])
            u1 = plsc.load_gather(ublk_v, [rr, iota + LANES, ul])
            i0 = plsc.load_gather(iblk_v, [rr, iota, il])
            i1 = plsc.load_gather(iblk_v, [rr, iota + LANES, il])
            p = u0 * i0 + u1 * i1

            row = (jj % GROUP_ITERS) * DEPTH + r
            pbuf_v[row] = p

            @pl.when(jj < niters - 1)
            def _():
                issue(j + DEPTH, r)

            if r == DEPTH - 1:
                @pl.when(jj % GROUP_ITERS == GROUP_ITERS - 1)
                def _():
                    acc = jnp.zeros((LANES,), jnp.float32)
                    for d in range(LANES):
                        acc = acc + plsc.load_gather(
                            pbuf_v, [iota, jnp.full((LANES,), d, jnp.int32)])
                    g0 = (jj - (GROUP_ITERS - 1)) * DEPTH
                    out_v[pl.ds(g0, LANES)] = acc

    pltpu.sync_copy(out_v, out_hbm.at[pl.ds(base, B_PER_W)])


def kernel(user_ids, item_ids, user_table, item_table):
    user_ids = user_ids.astype(jnp.int32)
    item_ids = item_ids.astype(jnp.int32)
    return _two_tower_sc(user_ids, item_ids, user_table.T, item_table.T)
